# baseline (speedup 1.0000x reference)
import functools

import jax
import jax.numpy as jnp
from jax import lax
from jax.experimental import pallas as pl
from jax.experimental.pallas import tpu as pltpu
from jax.experimental.pallas import tpu_sc as plsc

N = 8192
E = 16
K = 2
H = 78
HP = 128
RH = 1234
RHP = 1280
W0 = 30.0
T = 1024
MAXT = N * K // T + E
P = MAXT * T
RB = 1024
NB = N // RB

def _dot(a, b):
    return lax.dot_general(a, b, (((1,), (0,)), ((), ())),
                           preferred_element_type=jnp.float32)


def _router_body(ctx_ref, rw1_ref, rb1_ref, rw2_ref, rb2_ref,
                 topi_ref, gate_ref):
    h = jnp.maximum(_dot(ctx_ref[...], rw1_ref[...]) + rb1_ref[...], 0.0)
    lg = _dot(h, rw2_ref[...]) + rb2_ref[...]
    lane = lax.broadcasted_iota(jnp.int32, (RB, HP), 1)
    lg = jnp.where(lane < E, lg, -1e30)
    v0 = jnp.max(lg, axis=1, keepdims=True)
    i0 = jnp.argmax(lg, axis=1).astype(jnp.int32)
    lg2 = jnp.where(lane == i0[:, None], -1e30, lg)
    v1 = jnp.max(lg2, axis=1, keepdims=True)
    i1 = jnp.argmax(lg2, axis=1).astype(jnp.int32)
    g0 = 1.0 / (1.0 + jnp.exp(v1 - v0))
    topi_ref[:, 0:1] = i0[:, None]
    topi_ref[:, 1:2] = i1[:, None]
    gate_ref[:, 0:1] = g0
    gate_ref[:, 1:2] = 1.0 - g0


def _router(context, rw1p, rb1p, rw2p, rb2p):
    return pl.pallas_call(
        _router_body,
        grid=(NB,),
        in_specs=[
            pl.BlockSpec((RB, 64), lambda i: (i, 0)),
            pl.BlockSpec((64, RHP), lambda i: (0, 0)),
            pl.BlockSpec((1, RHP), lambda i: (0, 0)),
            pl.BlockSpec((RHP, HP), lambda i: (0, 0)),
            pl.BlockSpec((1, HP), lambda i: (0, 0)),
        ],
        out_specs=[
            pl.BlockSpec((RB, K), lambda i: (i, 0)),
            pl.BlockSpec((RB, K), lambda i: (i, 0)),
        ],
        out_shape=[
            jax.ShapeDtypeStruct((N, K), jnp.int32),
            jax.ShapeDtypeStruct((N, K), jnp.float32),
        ],
    )(context, rw1p, rb1p, rw2p, rb2p)


def _sc_gather(table, idx, D, nchunk):
    B = idx.shape[0]
    NW = 32
    bpw = B // NW
    csz = bpw // nchunk

    @functools.partial(
        pl.kernel,
        out_type=jax.ShapeDtypeStruct((B, D), jnp.float32),
        mesh=plsc.VectorSubcoreMesh(core_axis_name="c", subcore_axis_name="s"),
        scratch_types=[
            pltpu.VMEM((bpw,), jnp.int32),
            pltpu.VMEM((bpw, D), jnp.float32),
            pltpu.SemaphoreType.DMA,
        ],
    )
    def k(table_hbm, idx_hbm, out_hbm, idx_v, rows_v, sem):
        wid = lax.axis_index("s") * 2 + lax.axis_index("c")
        base = wid * bpw
        pltpu.sync_copy(idx_hbm.at[pl.ds(base, bpw)], idx_v)
        copies = []
        for c in range(nchunk):
            cp = pltpu.make_async_copy(
                table_hbm.at[idx_v.at[pl.ds(c * csz, csz)]],
                rows_v.at[pl.ds(c * csz, csz)], sem)
            cp.start()
            copies.append(cp)
        for cp in copies:
            cp.wait()
        pltpu.sync_copy(rows_v, out_hbm.at[pl.ds(base, bpw)])

    return k(table, idx)


def _sc_scatter_x(x128, ia, ib, nchunk):
    NW = 32
    tpw = N // NW
    csz = tpw // nchunk

    @functools.partial(
        pl.kernel,
        out_type=jax.ShapeDtypeStruct((P, HP), jnp.float32),
        mesh=plsc.VectorSubcoreMesh(core_axis_name="c", subcore_axis_name="s"),
        scratch_types=[
            pltpu.VMEM((nchunk, csz), jnp.int32),
            pltpu.VMEM((nchunk, csz), jnp.int32),
            pltpu.VMEM((tpw, HP), jnp.float32),
            pltpu.SemaphoreType.DMA,
        ],
    )
    def k(x_hbm, ia_hbm, ib_hbm, out_hbm, ia_v, ib_v, rows_v, sem):
        wid = lax.axis_index("s") * 2 + lax.axis_index("c")
        pltpu.sync_copy(x_hbm.at[pl.ds(wid * tpw, tpw)], rows_v)
        pltpu.sync_copy(ia_hbm.at[wid], ia_v)
        pltpu.sync_copy(ib_hbm.at[wid], ib_v)
        copies = []
        for c in range(nchunk):
            src = rows_v.at[pl.ds(c * csz, csz)]
            for idx_v in (ia_v, ib_v):
                cp = pltpu.make_async_copy(src, out_hbm.at[idx_v.at[c]], sem)
                cp.start()
                copies.append(cp)
        for cp in copies:
            cp.wait()

    return k(x128, ia, ib)


_INV_PI = 0.3183098861837907
_PI_HI = 3.140625
_PI_MID = 9.676535897932e-4
_PI_LO = float(3.141592653589793 - 3.140625 - jnp.float32(9.676535897932e-4))
_S1 = -1.66666666666666574e-1
_S2 = 8.33333333332249686e-3
_S3 = -1.98412698298579534e-4
_S4 = 2.75573192239858907e-6
_S5 = -2.50521083854417188e-8


def _fast_sin(z):
    k = jnp.floor(z * _INV_PI + 0.5)
    r = z - k * _PI_HI
    r = r - k * _PI_MID
    r = r - k * _PI_LO
    s = r * r
    p = r * (1.0 + s * (_S1 + s * (_S2 + s * (_S3 + s * (_S4 + s * _S5)))))
    ki = k.astype(jnp.int32)
    return jnp.where((ki & 1) == 0, p, -p)


def _expert_body(te_ref, nt_ref, x_ref, w0_ref, w1_ref, w2_ref, w3_ref, w4_ref,
                 o_ref):
    @pl.when(pl.program_id(0) < nt_ref[0])
    def _():
        lane = lax.broadcasted_iota(jnp.int32, (T, HP), 1)
        h = _fast_sin(W0 * _dot(x_ref[...], w0_ref[0]))
        for w_ref in (w1_ref, w2_ref, w3_ref):
            h = jnp.where(lane == H, 1.0, h)
            h = _fast_sin(W0 * _dot(h, w_ref[0]))
        h = jnp.where(lane == H, 1.0, h)
        o_ref[...] = _dot(h, w4_ref[0])


def _expert_mlp(tile_e, ntot, xg, w0a, w1a, w2a, w3a, w4a):
    grid_spec = pltpu.PrefetchScalarGridSpec(
        num_scalar_prefetch=2,
        grid=(MAXT,),
        in_specs=[
            pl.BlockSpec((T, HP), lambda t, te, nt: (t, 0)),
            pl.BlockSpec((1, HP, HP), lambda t, te, nt: (te[t], 0, 0)),
            pl.BlockSpec((1, HP, HP), lambda t, te, nt: (te[t], 0, 0)),
            pl.BlockSpec((1, HP, HP), lambda t, te, nt: (te[t], 0, 0)),
            pl.BlockSpec((1, HP, HP), lambda t, te, nt: (te[t], 0, 0)),
            pl.BlockSpec((1, HP, HP), lambda t, te, nt: (te[t], 0, 0)),
        ],
        out_specs=pl.BlockSpec((T, HP), lambda t, te, nt: (t, 0)),
    )
    return pl.pallas_call(
        _expert_body,
        grid_spec=grid_spec,
        out_shape=jax.ShapeDtypeStruct((P, HP), jnp.float32),
    )(tile_e, ntot, xg, w0a, w1a, w2a, w3a, w4a)


def _decoder_body(r0_ref, r1_ref, g_ref, dw1_ref, dw2_ref, o_ref):
    g = g_ref[...]
    f = g[:, 0:1] * r0_ref[...] + g[:, 1:2] * r1_ref[...]
    lane = lax.broadcasted_iota(jnp.int32, (RB, HP), 1)
    f = jnp.where(lane == H, 1.0, f)
    d1 = jnp.maximum(_dot(f, dw1_ref[...]), 0.0)
    d1 = jnp.where(lane == H, 1.0, d1)
    o_ref[...] = _dot(d1, dw2_ref[...])[:, 0:1]


def _decoder(rows_all, gates, dw1a, dw2row):
    return pl.pallas_call(
        _decoder_body,
        grid=(NB,),
        in_specs=[
            pl.BlockSpec((RB, HP), lambda i: (i, 0)),
            pl.BlockSpec((RB, HP), lambda i: (i + NB, 0)),
            pl.BlockSpec((RB, K), lambda i: (i, 0)),
            pl.BlockSpec((HP, HP), lambda i: (0, 0)),
            pl.BlockSpec((HP, HP), lambda i: (0, 0)),
        ],
        out_specs=pl.BlockSpec((RB, 1), lambda i: (i, 0)),
        out_shape=jax.ShapeDtypeStruct((N, 1), jnp.float32),
    )(rows_all, rows_all, gates, dw1a, dw2row)


def kernel(x, context, rW1, rb1, rW2, rb2, eW0, eb0, eW1, eb1, eW2, eb2,
           eW3, eb3, eW4, eb4, dW1, db1, dW2, db2):
    f32 = jnp.float32

    rw1p = jnp.zeros((64, RHP), f32).at[:, :RH].set(rW1)
    rb1p = jnp.zeros((1, RHP), f32).at[0, :RH].set(rb1)
    rw2p = jnp.zeros((RHP, HP), f32).at[:RH, :E].set(rW2)
    rb2p = jnp.zeros((1, HP), f32).at[0, :E].set(rb2)

    w0a = jnp.zeros((E, HP, HP), f32).at[:, :3, :H].set(eW0).at[:, 3, :H].set(eb0)
    def _aug(w, b):
        return jnp.zeros((E, HP, HP), f32).at[:, :H, :H].set(w).at[:, H, :H].set(b)
    w1a, w2a, w3a, w4a = _aug(eW1, eb1), _aug(eW2, eb2), _aug(eW3, eb3), _aug(eW4, eb4)

    dw1a = jnp.zeros((HP, HP), f32).at[:H, :H].set(dW1).at[H, :H].set(db1)
    dw2a = jnp.zeros((HP, HP), f32).at[:H, 0].set(dW2[:, 0]).at[H, 0].set(db2[0])

    x128 = jnp.zeros((N, HP), f32).at[:, :3].set(x).at[:, 3].set(1.0)

    topi, gates = _router(context, rw1p, rb1p, rw2p, rb2p)

    e_flat = topi.reshape(-1)
    onehot = (e_flat[:, None] == jnp.arange(E, dtype=jnp.int32)).astype(jnp.int32)
    ranks = jnp.sum((jnp.cumsum(onehot, axis=0) - onehot) * onehot,
                    axis=1).reshape(N, K)
    counts = jnp.sum(onehot, axis=0)
    ntiles = (counts + T - 1) // T
    tend = jnp.cumsum(ntiles)
    tbase = tend - ntiles
    pp = tbase[topi] * T + ranks
    tile_e = jnp.minimum(
        jnp.sum(jnp.arange(MAXT, dtype=jnp.int32)[:, None] >= tend[None, :],
                axis=1), E - 1).astype(jnp.int32)

    ia = pp[:, 0].reshape(32, 4, N // 32 // 4)
    ib = pp[:, 1].reshape(32, 4, N // 32 // 4)
    xg = _sc_scatter_x(x128, ia, ib, 4)

    out_pad = _expert_mlp(tile_e, tend[E - 1:E].astype(jnp.int32), xg,
                          w0a, w1a, w2a, w3a, w4a)

    idx2 = pp.T.reshape(-1)
    rows_all = _sc_gather(out_pad, idx2, HP, 8)

    return _decoder(rows_all, gates, dw1a, dw2a)


# speedup vs baseline: 1.0406x; 1.0406x over previous
import functools

import jax
import jax.numpy as jnp
from jax import lax
from jax.experimental import pallas as pl
from jax.experimental.pallas import tpu as pltpu
from jax.experimental.pallas import tpu_sc as plsc

N = 8192
E = 16
K = 2
H = 78
HP = 128
RH = 1234
RHP = 1280
W0 = 30.0
T = 512
MAXT = N * K // T + E
P = MAXT * T
RB = 1024
NB = N // RB

def _dot(a, b):
    return lax.dot_general(a, b, (((1,), (0,)), ((), ())),
                           preferred_element_type=jnp.float32)


def _router_body(ctx_ref, rw1_ref, rb1_ref, rw2_ref, rb2_ref,
                 topi_ref, gate_ref):
    h = jnp.maximum(_dot(ctx_ref[...], rw1_ref[...]) + rb1_ref[...], 0.0)
    lg = _dot(h, rw2_ref[...]) + rb2_ref[...]
    lane = lax.broadcasted_iota(jnp.int32, (RB, HP), 1)
    lg = jnp.where(lane < E, lg, -1e30)
    v0 = jnp.max(lg, axis=1, keepdims=True)
    i0 = jnp.argmax(lg, axis=1).astype(jnp.int32)
    lg2 = jnp.where(lane == i0[:, None], -1e30, lg)
    v1 = jnp.max(lg2, axis=1, keepdims=True)
    i1 = jnp.argmax(lg2, axis=1).astype(jnp.int32)
    g0 = 1.0 / (1.0 + jnp.exp(v1 - v0))
    topi_ref[:, 0:1] = i0[:, None]
    topi_ref[:, 1:2] = i1[:, None]
    gate_ref[:, 0:1] = g0
    gate_ref[:, 1:2] = 1.0 - g0


def _router(context, rw1p, rb1p, rw2p, rb2p):
    return pl.pallas_call(
        _router_body,
        grid=(NB,),
        in_specs=[
            pl.BlockSpec((RB, 64), lambda i: (i, 0)),
            pl.BlockSpec((64, RHP), lambda i: (0, 0)),
            pl.BlockSpec((1, RHP), lambda i: (0, 0)),
            pl.BlockSpec((RHP, HP), lambda i: (0, 0)),
            pl.BlockSpec((1, HP), lambda i: (0, 0)),
        ],
        out_specs=[
            pl.BlockSpec((RB, K), lambda i: (i, 0)),
            pl.BlockSpec((RB, K), lambda i: (i, 0)),
        ],
        out_shape=[
            jax.ShapeDtypeStruct((N, K), jnp.int32),
            jax.ShapeDtypeStruct((N, K), jnp.float32),
        ],
    )(context, rw1p, rb1p, rw2p, rb2p)


def _sc_gather(table, idx, D, nchunk):
    B = idx.shape[0]
    NW = 32
    bpw = B // NW
    csz = bpw // nchunk

    @functools.partial(
        pl.kernel,
        out_type=jax.ShapeDtypeStruct((B, D), jnp.float32),
        mesh=plsc.VectorSubcoreMesh(core_axis_name="c", subcore_axis_name="s"),
        scratch_types=[
            pltpu.VMEM((bpw,), jnp.int32),
            pltpu.VMEM((bpw, D), jnp.float32),
            pltpu.SemaphoreType.DMA,
        ],
    )
    def k(table_hbm, idx_hbm, out_hbm, idx_v, rows_v, sem):
        wid = lax.axis_index("s") * 2 + lax.axis_index("c")
        base = wid * bpw
        pltpu.sync_copy(idx_hbm.at[pl.ds(base, bpw)], idx_v)
        copies = []
        for c in range(nchunk):
            cp = pltpu.make_async_copy(
                table_hbm.at[idx_v.at[pl.ds(c * csz, csz)]],
                rows_v.at[pl.ds(c * csz, csz)], sem)
            cp.start()
            copies.append(cp)
        for cp in copies:
            cp.wait()
        pltpu.sync_copy(rows_v, out_hbm.at[pl.ds(base, bpw)])

    return k(table, idx)


def _sc_scatter_x(x128, ia, ib, nchunk):
    NW = 32
    tpw = N // NW
    csz = tpw // nchunk

    @functools.partial(
        pl.kernel,
        out_type=jax.ShapeDtypeStruct((P, HP), jnp.float32),
        mesh=plsc.VectorSubcoreMesh(core_axis_name="c", subcore_axis_name="s"),
        scratch_types=[
            pltpu.VMEM((nchunk, csz), jnp.int32),
            pltpu.VMEM((nchunk, csz), jnp.int32),
            pltpu.VMEM((tpw, HP), jnp.float32),
            pltpu.SemaphoreType.DMA,
        ],
    )
    def k(x_hbm, ia_hbm, ib_hbm, out_hbm, ia_v, ib_v, rows_v, sem):
        wid = lax.axis_index("s") * 2 + lax.axis_index("c")
        pltpu.sync_copy(x_hbm.at[pl.ds(wid * tpw, tpw)], rows_v)
        pltpu.sync_copy(ia_hbm.at[wid], ia_v)
        pltpu.sync_copy(ib_hbm.at[wid], ib_v)
        copies = []
        for c in range(nchunk):
            src = rows_v.at[pl.ds(c * csz, csz)]
            for idx_v in (ia_v, ib_v):
                cp = pltpu.make_async_copy(src, out_hbm.at[idx_v.at[c]], sem)
                cp.start()
                copies.append(cp)
        for cp in copies:
            cp.wait()

    return k(x128, ia, ib)


_INV_PI = 0.3183098861837907
_PI_HI = 3.140625
_PI_MID = 9.676535897932e-4
_PI_LO = float(3.141592653589793 - 3.140625 - jnp.float32(9.676535897932e-4))
_S1 = -1.66666666666666574e-1
_S2 = 8.33333333332249686e-3
_S3 = -1.98412698298579534e-4
_S4 = 2.75573192239858907e-6
_S5 = -2.50521083854417188e-8


def _fast_sin(z):
    k = jnp.floor(z * _INV_PI + 0.5)
    r = z - k * _PI_HI
    r = r - k * _PI_MID
    r = r - k * _PI_LO
    s = r * r
    p = r * (1.0 + s * (_S1 + s * (_S2 + s * (_S3 + s * (_S4 + s * _S5)))))
    ki = k.astype(jnp.int32)
    return jnp.where((ki & 1) == 0, p, -p)


def _expert_body(te_ref, nt_ref, x_ref, w0_ref, w1_ref, w2_ref, w3_ref, w4_ref,
                 o_ref):
    @pl.when(pl.program_id(0) < nt_ref[0])
    def _():
        lane = lax.broadcasted_iota(jnp.int32, (T, HP), 1)
        h = _fast_sin(W0 * _dot(x_ref[...], w0_ref[0]))
        for w_ref in (w1_ref, w2_ref, w3_ref):
            h = jnp.where(lane == H, 1.0, h)
            h = _fast_sin(W0 * _dot(h, w_ref[0]))
        h = jnp.where(lane == H, 1.0, h)
        o_ref[...] = _dot(h, w4_ref[0])


def _expert_mlp(tile_e, ntot, xg, w0a, w1a, w2a, w3a, w4a):
    grid_spec = pltpu.PrefetchScalarGridSpec(
        num_scalar_prefetch=2,
        grid=(MAXT,),
        in_specs=[
            pl.BlockSpec((T, HP), lambda t, te, nt: (t, 0)),
            pl.BlockSpec((1, HP, HP), lambda t, te, nt: (te[t], 0, 0)),
            pl.BlockSpec((1, HP, HP), lambda t, te, nt: (te[t], 0, 0)),
            pl.BlockSpec((1, HP, HP), lambda t, te, nt: (te[t], 0, 0)),
            pl.BlockSpec((1, HP, HP), lambda t, te, nt: (te[t], 0, 0)),
            pl.BlockSpec((1, HP, HP), lambda t, te, nt: (te[t], 0, 0)),
        ],
        out_specs=pl.BlockSpec((T, HP), lambda t, te, nt: (t, 0)),
    )
    return pl.pallas_call(
        _expert_body,
        grid_spec=grid_spec,
        out_shape=jax.ShapeDtypeStruct((P, HP), jnp.float32),
    )(tile_e, ntot, xg, w0a, w1a, w2a, w3a, w4a)


def _decoder_body(r0_ref, r1_ref, g_ref, dw1_ref, dw2_ref, o_ref):
    g = g_ref[...]
    f = g[:, 0:1] * r0_ref[...] + g[:, 1:2] * r1_ref[...]
    lane = lax.broadcasted_iota(jnp.int32, (RB, HP), 1)
    f = jnp.where(lane == H, 1.0, f)
    d1 = jnp.maximum(_dot(f, dw1_ref[...]), 0.0)
    d1 = jnp.where(lane == H, 1.0, d1)
    o_ref[...] = _dot(d1, dw2_ref[...])[:, 0:1]


def _decoder(rows_all, gates, dw1a, dw2row):
    return pl.pallas_call(
        _decoder_body,
        grid=(NB,),
        in_specs=[
            pl.BlockSpec((RB, HP), lambda i: (i, 0)),
            pl.BlockSpec((RB, HP), lambda i: (i + NB, 0)),
            pl.BlockSpec((RB, K), lambda i: (i, 0)),
            pl.BlockSpec((HP, HP), lambda i: (0, 0)),
            pl.BlockSpec((HP, HP), lambda i: (0, 0)),
        ],
        out_specs=pl.BlockSpec((RB, 1), lambda i: (i, 0)),
        out_shape=jax.ShapeDtypeStruct((N, 1), jnp.float32),
    )(rows_all, rows_all, gates, dw1a, dw2row)


def kernel(x, context, rW1, rb1, rW2, rb2, eW0, eb0, eW1, eb1, eW2, eb2,
           eW3, eb3, eW4, eb4, dW1, db1, dW2, db2):
    f32 = jnp.float32

    rw1p = jnp.zeros((64, RHP), f32).at[:, :RH].set(rW1)
    rb1p = jnp.zeros((1, RHP), f32).at[0, :RH].set(rb1)
    rw2p = jnp.zeros((RHP, HP), f32).at[:RH, :E].set(rW2)
    rb2p = jnp.zeros((1, HP), f32).at[0, :E].set(rb2)

    w0a = jnp.zeros((E, HP, HP), f32).at[:, :3, :H].set(eW0).at[:, 3, :H].set(eb0)
    def _aug(w, b):
        return jnp.zeros((E, HP, HP), f32).at[:, :H, :H].set(w).at[:, H, :H].set(b)
    w1a, w2a, w3a, w4a = _aug(eW1, eb1), _aug(eW2, eb2), _aug(eW3, eb3), _aug(eW4, eb4)

    dw1a = jnp.zeros((HP, HP), f32).at[:H, :H].set(dW1).at[H, :H].set(db1)
    dw2a = jnp.zeros((HP, HP), f32).at[:H, 0].set(dW2[:, 0]).at[H, 0].set(db2[0])

    x128 = jnp.zeros((N, HP), f32).at[:, :3].set(x).at[:, 3].set(1.0)

    topi, gates = _router(context, rw1p, rb1p, rw2p, rb2p)

    e_flat = topi.reshape(-1)
    onehot = (e_flat[:, None] == jnp.arange(E, dtype=jnp.int32)).astype(jnp.int32)
    ranks = jnp.sum((jnp.cumsum(onehot, axis=0) - onehot) * onehot,
                    axis=1).reshape(N, K)
    counts = jnp.sum(onehot, axis=0)
    ntiles = (counts + T - 1) // T
    tend = jnp.cumsum(ntiles)
    tbase = tend - ntiles
    pp = tbase[topi] * T + ranks
    tile_e = jnp.minimum(
        jnp.sum(jnp.arange(MAXT, dtype=jnp.int32)[:, None] >= tend[None, :],
                axis=1), E - 1).astype(jnp.int32)

    ia = pp[:, 0].reshape(32, 4, N // 32 // 4)
    ib = pp[:, 1].reshape(32, 4, N // 32 // 4)
    xg = _sc_scatter_x(x128, ia, ib, 4)

    out_pad = _expert_mlp(tile_e, tend[E - 1:E].astype(jnp.int32), xg,
                          w0a, w1a, w2a, w3a, w4a)

    idx2 = pp.T.reshape(-1)
    rows_all = _sc_gather(out_pad, idx2, HP, 8)

    return _decoder(rows_all, gates, dw1a, dw2a)
